# baseline (speedup 1.0000x reference)
import functools

import jax
import jax.numpy as jnp
from jax import lax
from jax.experimental import pallas as pl
from jax.experimental.pallas import tpu as pltpu
from jax.experimental.pallas import tpu_sc as plsc

N = 10000
NP = 10240
E = 320000
NW = 32
NB = 80
B = 128
TB = NW * NB
EP = TB * B
NB0 = 156
NB1 = 4
D = 128
G = 64
EPS = 1e-5
F32 = jnp.float32
HIGH = jax.lax.Precision.HIGHEST

_mesh = plsc.VectorSubcoreMesh(
    core_axis_name="c", subcore_axis_name="s", num_cores=2, num_subcores=16)

def _z16():
    return jnp.zeros((16,), F32)


def _deg_body(dst_hbm, ew_hbm, out_hbm, dst_v, ew_v, deg_l, part_v, stage_sh):
    c = lax.axis_index("c")
    s = lax.axis_index("s")
    wid = c * 16 + s
    pltpu.sync_copy(dst_hbm.at[pl.ds(wid * NB, NB)], dst_v)
    pltpu.sync_copy(ew_hbm.at[pl.ds(wid * NB, NB)], ew_v)

    def zero_body(i, carry):
        deg_l[pl.ds(i * 16, 16)] = _z16()
        return carry
    lax.fori_loop(0, NP // 16, zero_body, None)

    def batch_body(j, carry):
        for i in range(B // 16):
            sl = pl.ds(i * 16, 16)
            plsc.addupdate_scatter(deg_l, [dst_v[j, sl]], ew_v[j, sl])
        return carry
    lax.fori_loop(0, NB, batch_body, None)

    pltpu.sync_copy(deg_l, stage_sh.at[s])
    plsc.subcore_barrier()
    seg = NP // 16
    pltpu.sync_copy(stage_sh.at[:, pl.ds(s * seg, seg)], part_v)

    def red_body(m, carry):
        acc = part_v[0, pl.ds(m * 16, 16)]
        for k in range(1, 16):
            acc = acc + part_v[k, pl.ds(m * 16, 16)]
        deg_l[pl.ds(m * 16, 16)] = acc
        return carry
    lax.fori_loop(0, seg // 16, red_body, None)
    pltpu.sync_copy(deg_l.at[pl.ds(0, seg)], out_hbm.at[c, pl.ds(s * seg, seg)])


_SC_PARAMS = pltpu.CompilerParams(needs_layout_passes=False)

_deg_call = pl.kernel(
    _deg_body,
    out_type=jax.ShapeDtypeStruct((2, NP), F32),
    mesh=_mesh,
    compiler_params=_SC_PARAMS,
    scratch_types=[
        pltpu.VMEM((NB, B), jnp.int32),
        pltpu.VMEM((NB, B), F32),
        pltpu.VMEM((NP,), F32),
        pltpu.VMEM((16, NP // 16), F32),
        pltpu.VMEM_SHARED((16, NP), F32),
    ],
)


def _msg_body(C, y_hbm, src_hbm, dst_hbm, ew_hbm, out_hbm,
              src_r, dst_r, ew_r, rows0, rows1, s_sh, gsem, ssem, esem):
    c = lax.axis_index("c")
    s = lax.axis_index("s")
    nq = C // 16
    seg = NP // 16
    rows = (rows0, rows1)
    nb = jnp.where(c == 0, NB0, NB1)
    base = jnp.where(c == 0, s * NB0, 16 * NB0 + s * NB1)

    def zrow(i, carry):
        for q in range(nq):
            rows0[i, pl.ds(q * 16, 16)] = _z16()
        return carry
    lax.fori_loop(0, B, zrow, None)
    for k in range(seg // B):
        pltpu.sync_copy(rows0, s_sh.at[pl.ds(s * seg + k * B, B)])
    plsc.subcore_barrier()

    def fetch(j, q):
        pltpu.async_copy(src_hbm.at[base + j], src_r.at[q], esem)
        pltpu.async_copy(dst_hbm.at[base + j], dst_r.at[q], esem)
        pltpu.async_copy(ew_hbm.at[base + j], ew_r.at[q], esem)

    def scale(rows_v, q):
        def scale_body(eb, carry2):
            for u in range(8):
                e = eb * 8 + u
                ewb = plsc.load_gather(
                    ew_r, [jnp.full((16,), q, jnp.int32),
                           jnp.full((16,), e, jnp.int32)])
                for cc in range(nq):
                    sl = pl.ds(cc * 16, 16)
                    rows_v[e, sl] = rows_v[e, sl] * ewb
            return carry2
        lax.fori_loop(0, B // 8, scale_body, None)

    def gather(rows_v, q):
        return pltpu.async_copy(y_hbm.at[src_r.at[q]], rows_v, gsem)

    def scat(rows_v, q):
        return pltpu.async_copy(rows_v, s_sh.at[dst_r.at[q]], ssem,
                                add=True)

    for q0 in range(3):
        pltpu.sync_copy(src_hbm.at[base + q0], src_r.at[q0])
        pltpu.sync_copy(dst_hbm.at[base + q0], dst_r.at[q0])
        pltpu.sync_copy(ew_hbm.at[base + q0], ew_r.at[q0])
    gather(rows0, 0)

    def pipe_body(i, carry):
        for u in range(4):
            j = i * 4 + u
            r = u % 2
            qn = (u + 1) % 4

            def wait_scat(slot):
                pltpu.make_async_copy(rows[1 - r], s_sh.at[dst_r.at[slot]],
                                      ssem).wait()

            def wait_fetch(slot, jj):
                pltpu.make_async_copy(src_hbm.at[base + jj], src_r.at[slot],
                                      esem).wait()
                pltpu.make_async_copy(dst_hbm.at[base + jj], dst_r.at[slot],
                                      esem).wait()
                pltpu.make_async_copy(ew_hbm.at[base + jj], ew_r.at[slot],
                                      esem).wait()

            if u == 0:
                @pl.when(i > 0)
                def _():
                    wait_scat(3)
            else:
                wait_scat(u - 1)

            @pl.when(j + 3 < nb)
            def _():
                fetch(j + 3, (u + 3) % 4)

            if u < 2:
                @pl.when(i > 0)
                def _():
                    wait_fetch(qn, j + 1)
                gather(rows[1 - r], qn)
            elif u == 2:
                wait_fetch(qn, j + 1)
                gather(rows[1 - r], qn)
            else:
                @pl.when(j + 1 < nb)
                def _():
                    wait_fetch(qn, j + 1)
                    gather(rows[1 - r], qn)

            pltpu.make_async_copy(y_hbm.at[src_r.at[u]], rows[r],
                                  gsem).wait()
            scale(rows[r], u)
            scat(rows[r], u)
        return carry
    lax.fori_loop(0, nb // 4, pipe_body, None)
    pltpu.make_async_copy(rows1, s_sh.at[dst_r.at[3]], ssem).wait()

    plsc.subcore_barrier()
    for k in range(seg // B):
        sl = pl.ds(s * seg + k * B, B)
        pltpu.sync_copy(s_sh.at[sl], out_hbm.at[c, sl])


def _make_msg_call(C):
    return pl.kernel(
        functools.partial(_msg_body, C),
        out_type=jax.ShapeDtypeStruct((2, NP, C), F32),
        mesh=_mesh,
        compiler_params=_SC_PARAMS,
        scratch_types=[
            pltpu.VMEM((4, B), jnp.int32),
            pltpu.VMEM((4, B), jnp.int32),
            pltpu.VMEM((4, B), F32),
            pltpu.VMEM((B, C), F32),
            pltpu.VMEM((B, C), F32),
            pltpu.VMEM_SHARED((NP, C), F32),
            pltpu.SemaphoreType.DMA,
            pltpu.SemaphoreType.DMA,
            pltpu.SemaphoreType.DMA,
        ],
    )


_msg_call_128 = _make_msg_call(128)


def _tc1_body(degp_ref, x_ref, w1_ref, y_ref, dinv_ref):
    deg = degp_ref[0] + degp_ref[1] + 1.0
    dinv = jnp.where(deg > 0, lax.rsqrt(deg), 0.0)
    xw = jnp.dot(x_ref[...], w1_ref[...], precision=HIGH)
    y_ref[...] = xw * dinv
    dinv_ref[...] = dinv


_tc1_call = pl.pallas_call(
    _tc1_body,
    out_shape=[jax.ShapeDtypeStruct((NP, 128), F32),
               jax.ShapeDtypeStruct((NP, 1), F32)],
)


def _row_mask():
    return (lax.broadcasted_iota(jnp.int32, (NP, 1), 0) < N).astype(F32)


def _tc_mid_body(sp_ref, y_ref, dinv_ref, b_ref, g_ref, be_ref, w_ref,
                 ynext_ref):
    mask = _row_mask()
    h = sp_ref[0] + sp_ref[1] + y_ref[...]
    h = jnp.maximum(h * dinv_ref[...] + b_ref[...], 0.0) * mask
    mu = jnp.sum(h, axis=0, keepdims=True) * (1.0 / N)
    dcen = (h - mu) * mask
    var = jnp.sum(dcen * dcen, axis=0, keepdims=True) * (1.0 / N)
    hn = (dcen * lax.rsqrt(var + EPS) * g_ref[...] + be_ref[...]) * mask
    ynext_ref[...] = jnp.dot(hn, w_ref[...], precision=HIGH) * dinv_ref[...]


def _make_tc_mid(c_out):
    return pl.pallas_call(
        _tc_mid_body,
        out_shape=jax.ShapeDtypeStruct((NP, c_out), F32),
    )


_tc_mid_128 = _make_tc_mid(128)


def _tc3_body(sp_ref, y_ref, dinv_ref, b3_ref, batch_ref, wl_ref, bl_ref,
              out_ref):
    mask = _row_mask()
    h = sp_ref[0] + sp_ref[1] + y_ref[...]
    h = jnp.maximum(h * dinv_ref[...] + b3_ref[...], 0.0) * mask
    onehot = (batch_ref[...] ==
              lax.broadcasted_iota(jnp.int32, (G, NP), 0)).astype(F32)
    sums = jnp.dot(onehot, h, precision=HIGH)
    counts = jnp.sum(onehot, axis=1, keepdims=True)
    pooled = sums / jnp.maximum(counts, 1.0)
    out_ref[...] = jnp.dot(pooled, wl_ref[...], precision=HIGH) + bl_ref[...]


_tc3_call = pl.pallas_call(
    _tc3_body,
    out_shape=jax.ShapeDtypeStruct((G, 2), F32),
)


def kernel(x, edge_index, edge_attr, batch, W1, b1, g1, be1,
           W2, b2, g2, be2, W3, b3, Wl, bl):
    pad_e = EP - E
    src_p = jnp.concatenate(
        [edge_index[0], jnp.zeros((pad_e,), jnp.int32)]).reshape(TB, B)
    dst_p = jnp.concatenate(
        [edge_index[1], jnp.zeros((pad_e,), jnp.int32)]).reshape(TB, B)
    ew_p = jnp.concatenate(
        [edge_attr, jnp.zeros((pad_e,), F32)]).reshape(TB, B)
    x_p = jnp.concatenate([x, jnp.zeros((NP - N, D), F32)])
    batch_p = jnp.concatenate(
        [batch, jnp.full((NP - N,), G, jnp.int32)]).reshape(1, NP)

    W3p = jnp.concatenate([W3, jnp.zeros((128, 64), F32)], axis=1)
    b3p = jnp.concatenate([b3, jnp.zeros((64,), F32)])
    Wlp = jnp.concatenate([Wl, jnp.zeros((64, 2), F32)], axis=0)

    deg_part = _deg_call(dst_p, ew_p)
    y1, dinv = _tc1_call(deg_part.reshape(2, NP, 1), x_p, W1)
    s1 = _msg_call_128(y1, src_p, dst_p, ew_p)
    y2 = _tc_mid_128(s1, y1, dinv, b1.reshape(1, -1), g1.reshape(1, -1),
                     be1.reshape(1, -1), W2)
    s2 = _msg_call_128(y2, src_p, dst_p, ew_p)
    y3 = _tc_mid_128(s2, y2, dinv, b2.reshape(1, -1), g2.reshape(1, -1),
                     be2.reshape(1, -1), W3p)
    s3 = _msg_call_128(y3, src_p, dst_p, ew_p)
    return _tc3_call(s3, y3, dinv, b3p.reshape(1, -1), batch_p, Wlp,
                     bl.reshape(1, 2))


# speedup vs baseline: 1.1108x; 1.1108x over previous
import functools

import jax
import jax.numpy as jnp
from jax import lax
from jax.experimental import pallas as pl
from jax.experimental.pallas import tpu as pltpu
from jax.experimental.pallas import tpu_sc as plsc

N = 10000
NP = 10240
E = 320000
NW = 32
NB = 80
B = 128
TB = NW * NB
EP = TB * B
NB0 = 152
NB1 = 8
D = 128
G = 64
EPS = 1e-5
F32 = jnp.float32
HIGH = jax.lax.Precision.HIGHEST

_mesh = plsc.VectorSubcoreMesh(
    core_axis_name="c", subcore_axis_name="s", num_cores=2, num_subcores=16)

def _z16():
    return jnp.zeros((16,), F32)


def _deg_body(dst_hbm, ew_hbm, out_hbm, dst_v, ew_v, deg_l, part_v, stage_sh):
    c = lax.axis_index("c")
    s = lax.axis_index("s")
    wid = c * 16 + s
    pltpu.sync_copy(dst_hbm.at[pl.ds(wid * NB, NB)], dst_v)
    pltpu.sync_copy(ew_hbm.at[pl.ds(wid * NB, NB)], ew_v)

    def zero_body(i, carry):
        deg_l[pl.ds(i * 16, 16)] = _z16()
        return carry
    lax.fori_loop(0, NP // 16, zero_body, None)

    def batch_body(j, carry):
        for i in range(B // 16):
            sl = pl.ds(i * 16, 16)
            plsc.addupdate_scatter(deg_l, [dst_v[j, sl]], ew_v[j, sl])
        return carry
    lax.fori_loop(0, NB, batch_body, None)

    pltpu.sync_copy(deg_l, stage_sh.at[s])
    plsc.subcore_barrier()
    seg = NP // 16
    pltpu.sync_copy(stage_sh.at[:, pl.ds(s * seg, seg)], part_v)

    def red_body(m, carry):
        acc = part_v[0, pl.ds(m * 16, 16)]
        for k in range(1, 16):
            acc = acc + part_v[k, pl.ds(m * 16, 16)]
        deg_l[pl.ds(m * 16, 16)] = acc
        return carry
    lax.fori_loop(0, seg // 16, red_body, None)
    pltpu.sync_copy(deg_l.at[pl.ds(0, seg)], out_hbm.at[c, pl.ds(s * seg, seg)])


_SC_PARAMS = pltpu.CompilerParams(needs_layout_passes=False)

_deg_call = pl.kernel(
    _deg_body,
    out_type=jax.ShapeDtypeStruct((2, NP), F32),
    mesh=_mesh,
    compiler_params=_SC_PARAMS,
    scratch_types=[
        pltpu.VMEM((NB, B), jnp.int32),
        pltpu.VMEM((NB, B), F32),
        pltpu.VMEM((NP,), F32),
        pltpu.VMEM((16, NP // 16), F32),
        pltpu.VMEM_SHARED((16, NP), F32),
    ],
)


def _msg_body(C, y_hbm, src_hbm, dst_hbm, ew_hbm, out_hbm,
              src_r, dst_r, ew_r, rows0, rows1, s_sh, gsem, ssem, esem):
    c = lax.axis_index("c")
    s = lax.axis_index("s")
    nq = C // 16
    seg = NP // 16
    rows = (rows0, rows1)
    nb = jnp.where(c == 0, NB0, NB1)
    base = jnp.where(c == 0, s * NB0, 16 * NB0 + s * NB1)

    def zrow(i, carry):
        for q in range(nq):
            rows0[i, pl.ds(q * 16, 16)] = _z16()
        return carry
    lax.fori_loop(0, B, zrow, None)
    for k in range(seg // B):
        pltpu.sync_copy(rows0, s_sh.at[pl.ds(s * seg + k * B, B)])
    plsc.subcore_barrier()

    def fetch(j, q):
        pltpu.async_copy(src_hbm.at[base + j], src_r.at[q], esem)
        pltpu.async_copy(dst_hbm.at[base + j], dst_r.at[q], esem)
        pltpu.async_copy(ew_hbm.at[base + j], ew_r.at[q], esem)

    def scale(rows_v, q):
        def scale_body(eb, carry2):
            for u in range(8):
                e = eb * 8 + u
                ewb = plsc.load_gather(
                    ew_r, [jnp.full((16,), q, jnp.int32),
                           jnp.full((16,), e, jnp.int32)])
                for cc in range(nq):
                    sl = pl.ds(cc * 16, 16)
                    rows_v[e, sl] = rows_v[e, sl] * ewb
            return carry2
        lax.fori_loop(0, B // 8, scale_body, None)

    def gather(rows_v, q):
        return pltpu.async_copy(y_hbm.at[src_r.at[q]], rows_v, gsem)

    def scat(rows_v, q):
        return pltpu.async_copy(rows_v, s_sh.at[dst_r.at[q]], ssem,
                                add=True)

    for q0 in range(3):
        pltpu.sync_copy(src_hbm.at[base + q0], src_r.at[q0])
        pltpu.sync_copy(dst_hbm.at[base + q0], dst_r.at[q0])
        pltpu.sync_copy(ew_hbm.at[base + q0], ew_r.at[q0])
    gather(rows0, 0)

    def pipe_body(i, carry):
        for u in range(4):
            j = i * 4 + u
            r = u % 2
            qn = (u + 1) % 4

            def wait_scat(slot):
                pltpu.make_async_copy(rows[1 - r], s_sh.at[dst_r.at[slot]],
                                      ssem).wait()

            def wait_fetch(slot, jj):
                pltpu.make_async_copy(src_hbm.at[base + jj], src_r.at[slot],
                                      esem).wait()
                pltpu.make_async_copy(dst_hbm.at[base + jj], dst_r.at[slot],
                                      esem).wait()
                pltpu.make_async_copy(ew_hbm.at[base + jj], ew_r.at[slot],
                                      esem).wait()

            if u == 0:
                @pl.when(i > 0)
                def _():
                    wait_scat(3)
            else:
                wait_scat(u - 1)

            @pl.when(j + 3 < nb)
            def _():
                fetch(j + 3, (u + 3) % 4)

            if u < 2:
                @pl.when(i > 0)
                def _():
                    wait_fetch(qn, j + 1)
                gather(rows[1 - r], qn)
            elif u == 2:
                wait_fetch(qn, j + 1)
                gather(rows[1 - r], qn)
            else:
                @pl.when(j + 1 < nb)
                def _():
                    wait_fetch(qn, j + 1)
                    gather(rows[1 - r], qn)

            pltpu.make_async_copy(y_hbm.at[src_r.at[u]], rows[r],
                                  gsem).wait()
            scale(rows[r], u)
            scat(rows[r], u)
        return carry
    lax.fori_loop(0, nb // 4, pipe_body, None)
    pltpu.make_async_copy(rows1, s_sh.at[dst_r.at[3]], ssem).wait()

    plsc.subcore_barrier()
    for k in range(seg // B):
        sl = pl.ds(s * seg + k * B, B)
        pltpu.sync_copy(s_sh.at[sl], out_hbm.at[c, sl])


def _make_msg_call(C):
    return pl.kernel(
        functools.partial(_msg_body, C),
        out_type=jax.ShapeDtypeStruct((2, NP, C), F32),
        mesh=_mesh,
        compiler_params=_SC_PARAMS,
        scratch_types=[
            pltpu.VMEM((4, B), jnp.int32),
            pltpu.VMEM((4, B), jnp.int32),
            pltpu.VMEM((4, B), F32),
            pltpu.VMEM((B, C), F32),
            pltpu.VMEM((B, C), F32),
            pltpu.VMEM_SHARED((NP, C), F32),
            pltpu.SemaphoreType.DMA,
            pltpu.SemaphoreType.DMA,
            pltpu.SemaphoreType.DMA,
        ],
    )


_msg_call_128 = _make_msg_call(128)


def _tc1_body(degp_ref, x_ref, w1_ref, y_ref, dinv_ref):
    deg = degp_ref[0] + degp_ref[1] + 1.0
    dinv = jnp.where(deg > 0, lax.rsqrt(deg), 0.0)
    xw = jnp.dot(x_ref[...], w1_ref[...], precision=HIGH)
    y_ref[...] = xw * dinv
    dinv_ref[...] = dinv


_tc1_call = pl.pallas_call(
    _tc1_body,
    out_shape=[jax.ShapeDtypeStruct((NP, 128), F32),
               jax.ShapeDtypeStruct((NP, 1), F32)],
)


def _row_mask():
    return (lax.broadcasted_iota(jnp.int32, (NP, 1), 0) < N).astype(F32)


def _tc_mid_body(sp_ref, y_ref, dinv_ref, b_ref, g_ref, be_ref, w_ref,
                 ynext_ref):
    mask = _row_mask()
    h = sp_ref[0] + sp_ref[1] + y_ref[...]
    h = jnp.maximum(h * dinv_ref[...] + b_ref[...], 0.0) * mask
    mu = jnp.sum(h, axis=0, keepdims=True) * (1.0 / N)
    dcen = (h - mu) * mask
    var = jnp.sum(dcen * dcen, axis=0, keepdims=True) * (1.0 / N)
    hn = (dcen * lax.rsqrt(var + EPS) * g_ref[...] + be_ref[...]) * mask
    ynext_ref[...] = jnp.dot(hn, w_ref[...], precision=HIGH) * dinv_ref[...]


def _make_tc_mid(c_out):
    return pl.pallas_call(
        _tc_mid_body,
        out_shape=jax.ShapeDtypeStruct((NP, c_out), F32),
    )


_tc_mid_128 = _make_tc_mid(128)


def _tc3_body(sp_ref, y_ref, dinv_ref, b3_ref, batch_ref, wl_ref, bl_ref,
              out_ref):
    mask = _row_mask()
    h = sp_ref[0] + sp_ref[1] + y_ref[...]
    h = jnp.maximum(h * dinv_ref[...] + b3_ref[...], 0.0) * mask
    onehot = (batch_ref[...] ==
              lax.broadcasted_iota(jnp.int32, (G, NP), 0)).astype(F32)
    sums = jnp.dot(onehot, h, precision=HIGH)
    counts = jnp.sum(onehot, axis=1, keepdims=True)
    pooled = sums / jnp.maximum(counts, 1.0)
    out_ref[...] = jnp.dot(pooled, wl_ref[...], precision=HIGH) + bl_ref[...]


_tc3_call = pl.pallas_call(
    _tc3_body,
    out_shape=jax.ShapeDtypeStruct((G, 2), F32),
)


def kernel(x, edge_index, edge_attr, batch, W1, b1, g1, be1,
           W2, b2, g2, be2, W3, b3, Wl, bl):
    pad_e = EP - E
    src_p = jnp.concatenate(
        [edge_index[0], jnp.zeros((pad_e,), jnp.int32)]).reshape(TB, B)
    dst_p = jnp.concatenate(
        [edge_index[1], jnp.zeros((pad_e,), jnp.int32)]).reshape(TB, B)
    ew_p = jnp.concatenate(
        [edge_attr, jnp.zeros((pad_e,), F32)]).reshape(TB, B)
    x_p = jnp.concatenate([x, jnp.zeros((NP - N, D), F32)])
    batch_p = jnp.concatenate(
        [batch, jnp.full((NP - N,), G, jnp.int32)]).reshape(1, NP)

    W3p = jnp.concatenate([W3, jnp.zeros((128, 64), F32)], axis=1)
    b3p = jnp.concatenate([b3, jnp.zeros((64,), F32)])
    Wlp = jnp.concatenate([Wl, jnp.zeros((64, 2), F32)], axis=0)

    deg_part = _deg_call(dst_p, ew_p)
    y1, dinv = _tc1_call(deg_part.reshape(2, NP, 1), x_p, W1)
    s1 = _msg_call_128(y1, src_p, dst_p, ew_p)
    y2 = _tc_mid_128(s1, y1, dinv, b1.reshape(1, -1), g1.reshape(1, -1),
                     be1.reshape(1, -1), W2)
    s2 = _msg_call_128(y2, src_p, dst_p, ew_p)
    y3 = _tc_mid_128(s2, y2, dinv, b2.reshape(1, -1), g2.reshape(1, -1),
                     be2.reshape(1, -1), W3p)
    s3 = _msg_call_128(y3, src_p, dst_p, ew_p)
    return _tc3_call(s3, y3, dinv, b3p.reshape(1, -1), batch_p, Wlp,
                     bl.reshape(1, 2))
